# Initial kernel scaffold; baseline (speedup 1.0000x reference)
import jax
import jax.numpy as jnp
from jax.experimental import pallas as pl


def kernel(inputs, combinations, weights):
    raise NotImplementedError("write your pallas kernel here")


# speedup vs baseline: 4.4782x; 4.4782x over previous
import functools

import jax
import jax.numpy as jnp
from jax import lax
from jax.experimental import pallas as pl
from jax.experimental.pallas import tpu as pltpu
from jax.experimental.pallas import tpu_sc as plsc

N = 1024
D = 256
K = 512

_NC = 2
_NS = 16
_NW = _NC * _NS
_BPW = N // _NW


def _tc_body(x_ref, c_ref, w_ref, code_ref, oh_ref, p_ref):
    x = x_ref[...]
    w = w_ref[...]
    p = c_ref[...] / w
    p_ref[...] = p

    hi = jax.lax.Precision.HIGHEST
    xp = lax.dot_general(x, p, (((1,), (1,)), ((), ())),
                         preferred_element_type=jnp.float32, precision=hi)
    ones_row = jnp.ones((1, D), jnp.float32)
    pn = lax.dot_general(ones_row, p * p, (((1,), (1,)), ((), ())),
                         preferred_element_type=jnp.float32, precision=hi)
    s = pn - 2.0 * xp

    ki = lax.broadcasted_iota(jnp.int32, (N, K), 1)
    m1 = jnp.max(s, axis=1, keepdims=True)
    c1 = jnp.min(jnp.where(s == m1, ki, K), axis=1, keepdims=True)
    s2 = jnp.where(ki == c1, -jnp.inf, s)
    m2 = jnp.max(s2, axis=1, keepdims=True)
    c2 = jnp.min(jnp.where(s2 == m2, ki, K), axis=1, keepdims=True)

    oh1 = (ki == c1).astype(jnp.float32)
    oh2 = (ki == c2).astype(jnp.float32)
    p1 = lax.dot_general(oh1, p, (((1,), (0,)), ((), ())),
                         preferred_element_type=jnp.float32, precision=hi)
    p2 = lax.dot_general(oh2, p, (((1,), (0,)), ((), ())),
                         preferred_element_type=jnp.float32, precision=hi)
    r1 = jnp.sqrt(jnp.sum((x - p1) * (x - p1), axis=1, keepdims=True))
    r2 = jnp.sqrt(jnp.sum((x - p2) * (x - p2), axis=1, keepdims=True))
    take2 = (r2 > r1) | ((r2 == r1) & (c2 < c1))
    code = jnp.where(take2, c2, c1)

    code_ref[...] = code
    oh_ref[...] = (ki == code).astype(jnp.float32)


_tc_call = pl.pallas_call(
    _tc_body,
    out_shape=[
        jax.ShapeDtypeStruct((N, 1), jnp.int32),
        jax.ShapeDtypeStruct((N, K), jnp.float32),
        jax.ShapeDtypeStruct((K, D), jnp.float32),
    ],
)


@functools.cache
def _get_sc_gather():
    @functools.partial(
        pl.kernel,
        out_type=jax.ShapeDtypeStruct((N, D), jnp.float32),
        mesh=plsc.VectorSubcoreMesh(core_axis_name="c", subcore_axis_name="s"),
        scratch_types=[
            pltpu.VMEM((_BPW,), jnp.int32),
            pltpu.VMEM((_BPW, D), jnp.float32),
            pltpu.SemaphoreType.DMA,
        ],
    )
    def _sc_gather(p_hbm, idx_hbm, out_hbm, idx_v, rows_v, sem):
        wid = lax.axis_index("s") * _NC + lax.axis_index("c")
        base = wid * _BPW
        pltpu.sync_copy(idx_hbm.at[pl.ds(base, _BPW)], idx_v)
        pltpu.async_copy(p_hbm.at[idx_v], rows_v, sem).wait()
        pltpu.sync_copy(rows_v, out_hbm.at[pl.ds(base, _BPW)])

    return _sc_gather


def kernel(inputs, combinations, weights):
    shape = inputs.shape
    x = inputs.reshape(-1, shape[-1])
    code2d, one_hot, p = _tc_call(x, combinations, weights.reshape(1, -1))
    closest = _get_sc_gather()(p, code2d.reshape(N))
    return one_hot.reshape(shape[:-1] + (K,)), closest.reshape(shape)


# speedup vs baseline: 14.3869x; 3.2127x over previous
import functools

import jax
import jax.numpy as jnp
from jax import lax
from jax.experimental import pallas as pl
from jax.experimental.pallas import tpu as pltpu
from jax.experimental.pallas import tpu_sc as plsc

N = 1024
D = 256
K = 512

_NC = 2
_NS = 16
_NW = _NC * _NS
_BPW = N // _NW


def _tc_body(x_ref, c_ref, w_ref, code_ref, oh_ref, p_ref, cl_ref):
    x = x_ref[...]
    w = w_ref[...]
    p = c_ref[...] / w
    p_ref[...] = p

    hi = jax.lax.Precision.HIGHEST
    xp = lax.dot_general(x, p, (((1,), (1,)), ((), ())),
                         preferred_element_type=jnp.float32, precision=hi)
    ones_row = jnp.ones((1, D), jnp.float32)
    pn = lax.dot_general(ones_row, p * p, (((1,), (1,)), ((), ())),
                         preferred_element_type=jnp.float32, precision=hi)
    s = pn - 2.0 * xp

    ki = lax.broadcasted_iota(jnp.int32, (N, K), 1)
    m1 = jnp.max(s, axis=1, keepdims=True)
    c1 = jnp.min(jnp.where(s == m1, ki, K), axis=1, keepdims=True)
    s2 = jnp.where(ki == c1, -jnp.inf, s)
    m2 = jnp.max(s2, axis=1, keepdims=True)
    c2 = jnp.min(jnp.where(s2 == m2, ki, K), axis=1, keepdims=True)

    oh1 = (ki == c1).astype(jnp.float32)
    oh2 = (ki == c2).astype(jnp.float32)
    p1 = lax.dot_general(oh1, p, (((1,), (0,)), ((), ())),
                         preferred_element_type=jnp.float32, precision=hi)
    p2 = lax.dot_general(oh2, p, (((1,), (0,)), ((), ())),
                         preferred_element_type=jnp.float32, precision=hi)
    r1 = jnp.sqrt(jnp.sum((x - p1) * (x - p1), axis=1, keepdims=True))
    r2 = jnp.sqrt(jnp.sum((x - p2) * (x - p2), axis=1, keepdims=True))
    take2 = (r2 > r1) | ((r2 == r1) & (c2 < c1))
    code = jnp.where(take2, c2, c1)

    code_ref[...] = code
    oh_ref[...] = (ki == code).astype(jnp.float32)
    cl_ref[...] = jnp.where(take2, p2, p1)


_tc_call = pl.pallas_call(
    _tc_body,
    out_shape=[
        jax.ShapeDtypeStruct((N, 1), jnp.int32),
        jax.ShapeDtypeStruct((N, K), jnp.float32),
        jax.ShapeDtypeStruct((K, D), jnp.float32),
        jax.ShapeDtypeStruct((N, D), jnp.float32),
    ],
)


@functools.cache
def _get_sc_gather():
    @functools.partial(
        pl.kernel,
        out_type=jax.ShapeDtypeStruct((N, D), jnp.float32),
        mesh=plsc.VectorSubcoreMesh(core_axis_name="c", subcore_axis_name="s"),
        scratch_types=[
            pltpu.VMEM((_BPW,), jnp.int32),
            pltpu.VMEM((_BPW, D), jnp.float32),
            pltpu.SemaphoreType.DMA,
        ],
    )
    def _sc_gather(p_hbm, idx_hbm, out_hbm, idx_v, rows_v, sem):
        wid = lax.axis_index("s") * _NC + lax.axis_index("c")
        base = wid * _BPW
        pltpu.sync_copy(idx_hbm.at[pl.ds(base, _BPW)], idx_v)
        pltpu.async_copy(p_hbm.at[idx_v], rows_v, sem).wait()
        pltpu.sync_copy(rows_v, out_hbm.at[pl.ds(base, _BPW)])

    return _sc_gather


def kernel(inputs, combinations, weights):
    shape = inputs.shape
    x = inputs.reshape(-1, shape[-1])
    code2d, one_hot, p, closest = _tc_call(x, combinations, weights.reshape(1, -1))
    return one_hot.reshape(shape[:-1] + (K,)), closest.reshape(shape)
